# Initial kernel scaffold; baseline (speedup 1.0000x reference)
import jax
import jax.numpy as jnp
from jax.experimental import pallas as pl


def kernel(x, tables, offsets):
    raise NotImplementedError("write your pallas kernel here")


# speedup vs baseline: 13.3567x; 13.3567x over previous
import functools

import jax
import jax.numpy as jnp
import numpy as np
from jax import lax
from jax.experimental import pallas as pl
from jax.experimental.pallas import tpu as pltpu
from jax.experimental.pallas import tpu_sc as plsc

_F = 26
_V = 100000
_D = 16
_B = 4096
_NPAIR = (_F * (_F - 1)) // 2
_NW = 32
_IDXW = 128
_NBLK = _B // _IDXW
_NITEM = _NPAIR * _NBLK
_IPW = _NITEM // _NW
_C = 5
_NCHUNK = _IPW // _C


def _sc_ffm(idxa, idxb, table):
    mesh = plsc.VectorSubcoreMesh(core_axis_name="c", subcore_axis_name="s")

    @functools.partial(
        pl.kernel,
        mesh=mesh,
        out_type=jax.ShapeDtypeStruct((_NPAIR, _D, _B), jnp.float32),
        scratch_types=[
            pltpu.VMEM((_C * _IDXW,), jnp.int32),
            pltpu.VMEM((_C * _IDXW,), jnp.int32),
            pltpu.VMEM((_C * _IDXW, _D), jnp.float32),
            pltpu.VMEM((_C * _IDXW, _D), jnp.float32),
            pltpu.VMEM((_C * _D, _IDXW), jnp.float32),
            pltpu.SemaphoreType.DMA,
        ],
        compiler_params=pltpu.CompilerParams(
            use_tc_tiling_on_sc=False, needs_layout_passes=False),
    )
    def k(idxa_hbm, idxb_hbm, table_hbm, out_hbm,
          idxa_v, idxb_v, ra_v, rb_v, out_v, sem):
        wid = lax.axis_index("s") * 2 + lax.axis_index("c")
        item0 = wid * _IPW
        didx = lax.iota(jnp.int32, _D)

        def chunk(ci, carry):
            it0 = item0 + ci * _C
            pltpu.sync_copy(idxa_hbm.at[pl.ds(it0 * _IDXW, _C * _IDXW)], idxa_v)
            pltpu.sync_copy(idxb_hbm.at[pl.ds(it0 * _IDXW, _C * _IDXW)], idxb_v)
            cps = []
            for g in range(_C):
                cps.append(pltpu.async_copy(
                    table_hbm.at[idxa_v.at[pl.ds(g * _IDXW, _IDXW)]],
                    ra_v.at[pl.ds(g * _IDXW, _IDXW)], sem))
                cps.append(pltpu.async_copy(
                    table_hbm.at[idxb_v.at[pl.ds(g * _IDXW, _IDXW)]],
                    rb_v.at[pl.ds(g * _IDXW, _IDXW)], sem))
            for cp in cps:
                cp.wait()

            for g in range(_C):

                def prod4(q, c, g=g):
                    l = q * 4
                    for k_ in range(4):
                        m = g * _IDXW + l + k_
                        col = didx * 0 + (l + k_)
                        plsc.store_scatter(
                            out_v, [didx + g * _D, col], ra_v[m] * rb_v[m])
                    return c

                lax.fori_loop(0, _IDXW // 4, prod4, 0)

            for g in range(_C):
                it = it0 + g
                p = it // _NBLK
                blk = it % _NBLK
                pltpu.sync_copy(
                    out_v.at[pl.ds(g * _D, _D)],
                    out_hbm.at[p, :, pl.ds(blk * _IDXW, _IDXW)])
            return carry

        lax.fori_loop(0, _NCHUNK, chunk, 0)

    return k(idxa, idxb, table)


def kernel(x, tables, offsets):
    xi_t = (x + offsets[None, :]).T
    iu, ju = np.triu_indices(_F, k=1)
    iu = jnp.asarray(iu, jnp.int32)
    ju = jnp.asarray(ju, jnp.int32)
    idxa = (xi_t[iu] + (ju * _V)[:, None]).reshape(_NPAIR * _B)
    idxb = (xi_t[ju] + (iu * _V)[:, None]).reshape(_NPAIR * _B)
    table = tables.reshape(_F * _V, _D)
    out = _sc_ffm(idxa, idxb, table)
    return jnp.transpose(out, (2, 0, 1))


# speedup vs baseline: 15.4256x; 1.1549x over previous
import functools

import jax
import jax.numpy as jnp
import numpy as np
from jax import lax
from jax.experimental import pallas as pl
from jax.experimental.pallas import tpu as pltpu
from jax.experimental.pallas import tpu_sc as plsc

_F = 26
_V = 100000
_D = 16
_B = 4096
_NPAIR = (_F * (_F - 1)) // 2
_NW = 32
_IDXW = 128
_NBLK = _B // _IDXW
_NITEM = _NPAIR * _NBLK
_IPW = _NITEM // _NW
_C = 5
_NCHUNK = _IPW // _C
_CW = _C * _IDXW


def _sc_ffm(idxa, idxb, table):
    mesh = plsc.VectorSubcoreMesh(core_axis_name="c", subcore_axis_name="s")

    @functools.partial(
        pl.kernel,
        mesh=mesh,
        out_type=jax.ShapeDtypeStruct((_NPAIR, _D, _B), jnp.float32),
        scratch_types=[
            pltpu.VMEM((2, _CW), jnp.int32),
            pltpu.VMEM((2, _CW), jnp.int32),
            pltpu.VMEM((2 * _CW, _D), jnp.float32),
            pltpu.VMEM((2 * _CW, _D), jnp.float32),
            pltpu.VMEM((2 * _C * _D, _IDXW), jnp.float32),
            pltpu.SemaphoreType.DMA,
            pltpu.SemaphoreType.DMA,
            pltpu.SemaphoreType.DMA,
            pltpu.SemaphoreType.DMA,
            pltpu.SemaphoreType.DMA,
        ],
        compiler_params=pltpu.CompilerParams(
            use_tc_tiling_on_sc=False, needs_layout_passes=False),
    )
    def k(idxa_hbm, idxb_hbm, table_hbm, out_hbm,
          idxa_v, idxb_v, ra_v, rb_v, out_v, semi, semg0, semg1, semo0, semo1):
        wid = lax.axis_index("s") * 2 + lax.axis_index("c")
        item0 = wid * _IPW
        didx = lax.iota(jnp.int32, _D)
        semg = (semg0, semg1)
        semo = (semo0, semo1)

        def idx_copy(cj, slot, fire):
            off = (item0 + cj * _C) * _IDXW
            for src, dst in ((idxa_hbm, idxa_v), (idxb_hbm, idxb_v)):
                if fire:
                    pltpu.async_copy(
                        src.at[pl.ds(off, _CW)], dst.at[slot], semi)
                else:
                    pltpu.make_async_copy(
                        src.at[pl.ds(off, _CW)], dst.at[slot], semi).wait()

        def gathers(slot, fire):
            for g in range(_C):
                s = pl.ds(g * _IDXW, _IDXW)
                d = pl.ds(slot * _CW + g * _IDXW, _IDXW)
                for iv, rv in ((idxa_v, ra_v), (idxb_v, rb_v)):
                    cp = pltpu.make_async_copy(
                        table_hbm.at[iv.at[slot].at[s]], rv.at[d], semg[slot])
                    if fire:
                        cp.start()
                    else:
                        cp.wait()

        def out_dma(cj, slot, fire):
            for g in range(_C):
                it = item0 + cj * _C + g
                p = it // _NBLK
                blk = it % _NBLK
                cp = pltpu.make_async_copy(
                    out_v.at[pl.ds((slot * _C + g) * _D, _D)],
                    out_hbm.at[p, :, pl.ds(blk * _IDXW, _IDXW)],
                    semo[slot])
                if fire:
                    cp.start()
                else:
                    cp.wait()

        def compute(slot):
            for g in range(_C):
                row0 = (slot * _C + g) * _D
                rbase = slot * _CW + g * _IDXW

                def prod4(q, c, row0=row0, rbase=rbase):
                    l = q * 4
                    for k_ in range(4):
                        m = rbase + l + k_
                        col = didx * 0 + (l + k_)
                        plsc.store_scatter(
                            out_v, [didx + row0, col], ra_v[m] * rb_v[m])
                    return c

                lax.fori_loop(0, _IDXW // 4, prod4, 0)

        idx_copy(0, 0, True)
        idx_copy(1, 1, True)
        idx_copy(0, 0, False)
        gathers(0, True)

        def body(ci, carry):
            for s_ in (0, 1):
                @pl.when(ci % 2 == s_)
                def _(s_=s_):
                    cur, oth = s_, 1 - s_

                    @pl.when(ci + 1 < _NCHUNK)
                    def _():
                        idx_copy(ci + 1, oth, False)
                        gathers(oth, True)

                    gathers(cur, False)
                    @pl.when(ci + 2 < _NCHUNK)
                    def _():
                        idx_copy(ci + 2, cur, True)

                    @pl.when(ci >= 2)
                    def _():
                        out_dma(ci - 2, cur, False)

                    compute(cur)
                    out_dma(ci, cur, True)
            return carry

        lax.fori_loop(0, _NCHUNK, body, 0)

        out_dma(_NCHUNK - 2, (_NCHUNK - 2) % 2, False)
        out_dma(_NCHUNK - 1, (_NCHUNK - 1) % 2, False)

    return k(idxa, idxb, table)


def kernel(x, tables, offsets):
    xi_t = (x + offsets[None, :]).T
    iu, ju = np.triu_indices(_F, k=1)
    iu = jnp.asarray(iu, jnp.int32)
    ju = jnp.asarray(ju, jnp.int32)
    idxa = (xi_t[iu] + (ju * _V)[:, None]).reshape(_NPAIR * _B)
    idxb = (xi_t[ju] + (iu * _V)[:, None]).reshape(_NPAIR * _B)
    table = tables.reshape(_F * _V, _D)
    out = _sc_ffm(idxa, idxb, table)
    return jnp.transpose(out, (2, 0, 1))
